# Initial kernel scaffold; baseline (speedup 1.0000x reference)
import jax
import jax.numpy as jnp
from jax.experimental import pallas as pl


def kernel(x, params):
    raise NotImplementedError("write your pallas kernel here")


# speedup vs baseline: 1.0552x; 1.0552x over previous
import jax
import jax.numpy as jnp
from jax import lax
from jax.experimental import pallas as pl

EPS = 1e-5


def _conv(x, w, b, stride=(1, 1), padding=((0, 0), (0, 0))):
    out = lax.conv_general_dilated(x, w, window_strides=stride, padding=padding,
                                   dimension_numbers=('NCHW', 'OIHW', 'NCHW'))
    return out + b[None, :, None, None]


def _conv_t(x, w, b, stride, kernel, padding, out_pad):
    kh, kw = kernel
    ph, pw = padding
    oph, opw = out_pad
    pads = ((kh - 1 - ph, kh - 1 - ph + oph), (kw - 1 - pw, kw - 1 - pw + opw))
    out = lax.conv_general_dilated(x, w, window_strides=(1, 1), padding=pads,
                                   lhs_dilation=stride,
                                   dimension_numbers=('NCHW', 'OIHW', 'NCHW'))
    return out + b[None, :, None, None]


def _bn(x, g, b):
    m = x.mean(axis=(0, 2, 3), keepdims=True)
    v = x.var(axis=(0, 2, 3), keepdims=True)
    return g[None, :, None, None] * (x - m) * lax.rsqrt(v + EPS) + b[None, :, None, None]


def _res(x, w1, b1, w2, b2):
    h = jax.nn.relu(x)
    h = _conv(h, w1, b1, (1, 1), ((1, 1), (1, 1)))
    h = jax.nn.relu(h)
    h = _conv(h, w2, b2)
    return x + h


def _vq_body(zf_ref, e_ref, et_ref, q_ref, loss_ref):
    f = zf_ref[:]
    E = e_ref[:]
    e2 = jnp.sum(E * E, axis=0, keepdims=True)
    scores = e2 - 2.0 * jnp.dot(f, E, preferred_element_type=jnp.float32)
    idx = jnp.argmin(scores, axis=1)
    onehot = (lax.broadcasted_iota(jnp.int32, scores.shape, 1)
              == idx[:, None]).astype(jnp.float32)
    q = jnp.dot(onehot, et_ref[:], preferred_element_type=jnp.float32)
    q_ref[:] = q
    d = q - f
    part = jnp.sum(d * d).reshape(1, 1)

    @pl.when(pl.program_id(0) == 0)
    def _():
        loss_ref[...] = jnp.zeros((1, 1), jnp.float32)

    loss_ref[...] += part


def _vq_pallas(zf, E):
    N, D = zf.shape
    K = E.shape[1]
    M = 512
    assert N % M == 0
    grid = (N // M,)
    quant, loss_sum = pl.pallas_call(
        _vq_body,
        grid=grid,
        in_specs=[
            pl.BlockSpec((M, D), lambda i: (i, 0)),
            pl.BlockSpec((D, K), lambda i: (0, 0)),
            pl.BlockSpec((K, D), lambda i: (0, 0)),
        ],
        out_specs=[
            pl.BlockSpec((M, D), lambda i: (i, 0)),
            pl.BlockSpec((1, 1), lambda i: (0, 0)),
        ],
        out_shape=[
            jax.ShapeDtypeStruct((N, D), jnp.float32),
            jax.ShapeDtypeStruct((1, 1), jnp.float32),
        ],
    )(zf, E, E.T)
    return quant, loss_sum[0, 0]


def kernel(x, params):
    p = params
    h = _conv(x, p['enc_w1'], p['enc_b1'], (2, 2), ((1, 1), (1, 1)))
    h = jax.nn.relu(_bn(h, p['enc_g1'], p['enc_be1']))
    h = _conv(h, p['enc_w2'], p['enc_b2'], (2, 2), ((1, 1), (1, 1)))
    h = jax.nn.relu(_bn(h, p['enc_g2'], p['enc_be2']))
    h = _conv(h, p['enc_w3'], p['enc_b3'])
    h = _conv(h, p['pre_w1'], p['pre_b1'])
    h = _res(h, p['pre_r1_w1'], p['pre_r1_b1'], p['pre_r1_w2'], p['pre_r1_b2'])
    h = _res(h, p['pre_r2_w1'], p['pre_r2_b1'], p['pre_r2_w2'], p['pre_r2_b2'])
    z = _conv(h, p['pre_w2'], p['pre_b2'])

    E = p['embedding']
    B, D, H, W = z.shape
    zf = jnp.transpose(z, (0, 2, 3, 1)).reshape(-1, D)
    quant_f, loss_sum = _vq_pallas(zf, E)
    loss = 1.25 * loss_sum / zf.size
    zq = jnp.transpose(quant_f.reshape(B, H, W, D), (0, 3, 1, 2))

    h = _conv(zq, p['post_w1'], p['post_b1'])
    h = _res(h, p['post_r1_w1'], p['post_r1_b1'], p['post_r1_w2'], p['post_r1_b2'])
    h = _res(h, p['post_r2_w1'], p['post_r2_b1'], p['post_r2_w2'], p['post_r2_b2'])
    h = _conv(h, p['post_w2'], p['post_b2'])
    h = _conv_t(h, p['dec_w1'], p['dec_b1'], (2, 2), (4, 3), (1, 1), (0, 0))
    h = jax.nn.relu(_bn(h, p['dec_g1'], p['dec_be1']))
    recon = _conv_t(h, p['dec_w2'], p['dec_b2'], (2, 2), (4, 3), (1, 1), (0, 1))
    return recon, loss


# speedup vs baseline: 1.1251x; 1.0662x over previous
import jax
import jax.numpy as jnp
from jax import lax
from jax.experimental import pallas as pl

EPS = 1e-5


def _conv(x, w, b, stride=(1, 1), padding=((0, 0), (0, 0))):
    out = lax.conv_general_dilated(x, w, window_strides=stride, padding=padding,
                                   dimension_numbers=('NCHW', 'OIHW', 'NCHW'))
    return out + b[None, :, None, None]


def _conv_t(x, w, b, stride, kernel, padding, out_pad):
    kh, kw = kernel
    ph, pw = padding
    oph, opw = out_pad
    pads = ((kh - 1 - ph, kh - 1 - ph + oph), (kw - 1 - pw, kw - 1 - pw + opw))
    out = lax.conv_general_dilated(x, w, window_strides=(1, 1), padding=pads,
                                   lhs_dilation=stride,
                                   dimension_numbers=('NCHW', 'OIHW', 'NCHW'))
    return out + b[None, :, None, None]


def _bn(x, g, b):
    m = x.mean(axis=(0, 2, 3), keepdims=True)
    v = x.var(axis=(0, 2, 3), keepdims=True)
    return g[None, :, None, None] * (x - m) * lax.rsqrt(v + EPS) + b[None, :, None, None]


def _res(x, w1, b1, w2, b2):
    h = jax.nn.relu(x)
    h = _conv(h, w1, b1, (1, 1), ((1, 1), (1, 1)))
    h = jax.nn.relu(h)
    h = _conv(h, w2, b2)
    return x + h


def _vq_body(h_ref, e_ref, et_ref, w2_ref, b2_ref, wp_ref, bp_ref,
             out_ref, loss_ref):
    h = h_ref[0]
    z = jnp.dot(w2_ref[:], h, preferred_element_type=jnp.float32) + b2_ref[:]
    et = et_ref[:]
    e2 = jnp.sum(et * et, axis=1, keepdims=True)
    scores = e2 - 2.0 * jnp.dot(et, z, preferred_element_type=jnp.float32)
    idx = jnp.argmin(scores, axis=0)
    onehot = (lax.broadcasted_iota(jnp.int32, scores.shape, 0)
              == idx[None, :]).astype(jnp.float32)
    quant = jnp.dot(e_ref[:], onehot, preferred_element_type=jnp.float32)
    d = quant - z
    part = jnp.sum(d * d).reshape(1, 1)
    out_ref[0] = jnp.dot(wp_ref[:], quant,
                         preferred_element_type=jnp.float32) + bp_ref[:]

    @pl.when(pl.program_id(0) == 0)
    def _():
        loss_ref[...] = jnp.zeros((1, 1), jnp.float32)

    loss_ref[...] += part


def _vq_pallas(h, E, w2, b2, wp, bp):
    B, D, S = h.shape
    K = E.shape[1]
    grid = (B,)
    out, loss_sum = pl.pallas_call(
        _vq_body,
        grid=grid,
        in_specs=[
            pl.BlockSpec((1, D, S), lambda i: (i, 0, 0)),
            pl.BlockSpec((D, K), lambda i: (0, 0)),
            pl.BlockSpec((K, D), lambda i: (0, 0)),
            pl.BlockSpec((D, D), lambda i: (0, 0)),
            pl.BlockSpec((D, 1), lambda i: (0, 0)),
            pl.BlockSpec((D, D), lambda i: (0, 0)),
            pl.BlockSpec((D, 1), lambda i: (0, 0)),
        ],
        out_specs=[
            pl.BlockSpec((1, D, S), lambda i: (i, 0, 0)),
            pl.BlockSpec((1, 1), lambda i: (0, 0)),
        ],
        out_shape=[
            jax.ShapeDtypeStruct((B, D, S), jnp.float32),
            jax.ShapeDtypeStruct((1, 1), jnp.float32),
        ],
    )(h, E, E.T, w2, b2, wp, bp)
    return out, loss_sum[0, 0]


def kernel(x, params):
    p = params
    h = _conv(x, p['enc_w1'], p['enc_b1'], (2, 2), ((1, 1), (1, 1)))
    h = jax.nn.relu(_bn(h, p['enc_g1'], p['enc_be1']))
    h = _conv(h, p['enc_w2'], p['enc_b2'], (2, 2), ((1, 1), (1, 1)))
    h = jax.nn.relu(_bn(h, p['enc_g2'], p['enc_be2']))
    h = _conv(h, p['enc_w3'], p['enc_b3'])
    h = _conv(h, p['pre_w1'], p['pre_b1'])
    h = _res(h, p['pre_r1_w1'], p['pre_r1_b1'], p['pre_r1_w2'], p['pre_r1_b2'])
    h = _res(h, p['pre_r2_w1'], p['pre_r2_b1'], p['pre_r2_w2'], p['pre_r2_b2'])

    E = p['embedding']
    B, D, H, W = h.shape
    out, loss_sum = _vq_pallas(
        h.reshape(B, D, H * W), E,
        p['pre_w2'][:, :, 0, 0], p['pre_b2'][:, None],
        p['post_w1'][:, :, 0, 0], p['post_b1'][:, None])
    loss = 1.25 * loss_sum / (B * D * H * W)
    h = out.reshape(B, D, H, W)

    h = _res(h, p['post_r1_w1'], p['post_r1_b1'], p['post_r1_w2'], p['post_r1_b2'])
    h = _res(h, p['post_r2_w1'], p['post_r2_b1'], p['post_r2_w2'], p['post_r2_b2'])
    h = _conv(h, p['post_w2'], p['post_b2'])
    h = _conv_t(h, p['dec_w1'], p['dec_b1'], (2, 2), (4, 3), (1, 1), (0, 0))
    h = jax.nn.relu(_bn(h, p['dec_g1'], p['dec_be1']))
    recon = _conv_t(h, p['dec_w2'], p['dec_b2'], (2, 2), (4, 3), (1, 1), (0, 1))
    return recon, loss
